# baseline (speedup 1.0000x reference)
import jax
import jax.numpy as jnp
from jax.experimental import pallas as pl
from jax.experimental.pallas import tpu as pltpu

N = 8192
INPUT_D = 256
HIDDEN_D = 128
OUTPUT_D = 64
ALP = 0.5
LAM = 1.0

BM = 256
P = N // BM
CK = 1024
NC = N // CK
R = P // NC

_BAND_OFF = []
_off = 0
for _b in range(NC):
    _BAND_OFF.append(_off)
    _off += R * (NC - _b)
SWEEP2_STEPS = _off


def _decode(u):
    i = jnp.int32(0)
    c = jnp.int32(0)
    for b in range(NC):
        size = R * (NC - b)
        v = u - _BAND_OFF[b]
        within = jnp.logical_and(v >= 0, v < size)
        i = jnp.where(within, b * R + v // (NC - b), i)
        c = jnp.where(within, b + v % (NC - b), c)
    return i, c


def _h_kernel(x_ref, w_ref, b_ref, h_ref):
    h_ref[...] = (
        jnp.dot(x_ref[...], w_ref[...], preferred_element_type=jnp.float32)
        + b_ref[...]
    )


def _fused_kernel(
    h_ref, a1_ref, d_ref, a2_ref, w2_ref, b2_ref,
    out_ref, y1_scr, dh_scr, acc2_scr,
):
    t = pl.program_id(0)

    @pl.when(t == 0)
    def _():
        y1_scr[...] = jnp.zeros_like(y1_scr)

    @pl.when(t < P)
    def _():
        h = h_ref[...].astype(jnp.bfloat16)
        a = a1_ref[...].astype(jnp.bfloat16)
        ah = jnp.dot(a, h, preferred_element_type=jnp.float32)
        dh = jnp.dot(
            d_ref[...].astype(jnp.bfloat16), h, preferred_element_type=jnp.float32
        )
        acc2_scr[pl.ds(t * BM, BM), :] = jnp.dot(
            a, y1_scr[...].astype(jnp.bfloat16), preferred_element_type=jnp.float32
        )
        rows = pl.ds(t * BM, BM)
        dh_scr[rows, :] = dh
        y1_scr[rows, :] = (
            (1.0 - ALP) * h_ref[rows, :] + (ALP * LAM) * ah + ALP * dh
        )

    @pl.when(t >= P)
    def _():
        i, c = _decode(t - P)
        y1c = y1_scr[pl.ds(c * CK, CK), :]
        gid = c * CK + jax.lax.broadcasted_iota(jnp.int32, (CK, 1), 0)
        y1m = jnp.where(gid >= i * BM, y1c, 0.0).astype(jnp.bfloat16)
        rows = pl.ds(i * BM, BM)
        acc2_scr[rows, :] += jnp.dot(
            a2_ref[...].astype(jnp.bfloat16), y1m, preferred_element_type=jnp.float32
        )

        @pl.when(c == NC - 1)
        def _():
            y2 = (
                (1.0 - ALP) * y1_scr[rows, :]
                + (ALP * LAM) * acc2_scr[rows, :]
                + ALP * dh_scr[rows, :]
            )
            z = jnp.maximum(y2, 0.0)
            out_ref[...] = (
                jnp.dot(z, w2_ref[...], preferred_element_type=jnp.float32)
                + b2_ref[...]
            )


def _a1_map(t):
    return (jnp.minimum(t, P - 1), 0)


def _a2_map(t):
    i, c = _decode(jnp.maximum(t - P, 0))
    return (i, c)


def _out_map(t):
    i, _ = _decode(jnp.maximum(t - P, 0))
    return (i, 0)


def kernel(x, sem_adj, norm_diag, W_bef, b_bef, W_aft, b_aft):
    h = pl.pallas_call(
        _h_kernel,
        out_shape=jax.ShapeDtypeStruct((N, HIDDEN_D), jnp.float32),
    )(x, W_bef, b_bef.reshape(1, HIDDEN_D))

    out = pl.pallas_call(
        _fused_kernel,
        grid=(P + SWEEP2_STEPS,),
        in_specs=[
            pl.BlockSpec((N, HIDDEN_D), lambda t: (0, 0)),
            pl.BlockSpec((BM, N), _a1_map),
            pl.BlockSpec((BM, N), _a1_map),
            pl.BlockSpec((BM, CK), _a2_map),
            pl.BlockSpec((HIDDEN_D, OUTPUT_D), lambda t: (0, 0)),
            pl.BlockSpec((1, OUTPUT_D), lambda t: (0, 0)),
        ],
        out_specs=pl.BlockSpec((BM, OUTPUT_D), _out_map),
        out_shape=jax.ShapeDtypeStruct((N, OUTPUT_D), jnp.float32),
        scratch_shapes=[
            pltpu.VMEM((N, HIDDEN_D), jnp.float32),
            pltpu.VMEM((N, HIDDEN_D), jnp.float32),
            pltpu.VMEM((N, HIDDEN_D), jnp.float32),
        ],
        compiler_params=pltpu.CompilerParams(
            dimension_semantics=("arbitrary",),
        ),
    )(h, sem_adj, norm_diag, sem_adj, W_aft, b_aft.reshape(1, OUTPUT_D))

    return out


# speedup vs baseline: 1.1355x; 1.1355x over previous
import jax
import jax.numpy as jnp
from jax.experimental import pallas as pl
from jax.experimental.pallas import tpu as pltpu

N = 8192
INPUT_D = 256
HIDDEN_D = 128
OUTPUT_D = 64
ALP = 0.5
LAM = 1.0

BM = 256
P = N // BM
CK = 2048
NC = N // CK
R = P // NC

_BAND_OFF = []
_off = 0
for _b in range(NC):
    _BAND_OFF.append(_off)
    _off += R * (NC - _b)
SWEEP2_STEPS = _off


def _decode(u):
    i = jnp.int32(0)
    c = jnp.int32(0)
    for b in range(NC):
        size = R * (NC - b)
        v = u - _BAND_OFF[b]
        within = jnp.logical_and(v >= 0, v < size)
        i = jnp.where(within, b * R + v // (NC - b), i)
        c = jnp.where(within, b + v % (NC - b), c)
    return i, c


def _h_kernel(x_ref, w_ref, b_ref, h_ref):
    h_ref[...] = (
        jnp.dot(x_ref[...], w_ref[...], preferred_element_type=jnp.float32)
        + b_ref[...]
    )


def _fused_kernel(
    h_ref, a1_ref, d_ref, a2_ref, w2_ref, b2_ref,
    out_ref, y1_scr, dh_scr, acc2_scr,
):
    t = pl.program_id(0)

    @pl.when(t == 0)
    def _():
        y1_scr[...] = jnp.zeros_like(y1_scr)

    @pl.when(t < P)
    def _():
        h = h_ref[...].astype(jnp.bfloat16)
        a = a1_ref[...].astype(jnp.bfloat16)
        ah = jnp.dot(a, h, preferred_element_type=jnp.float32)
        dh = jnp.dot(
            d_ref[...].astype(jnp.bfloat16), h, preferred_element_type=jnp.float32
        )
        acc2_scr[pl.ds(t * BM, BM), :] = jnp.dot(
            a, y1_scr[...].astype(jnp.bfloat16), preferred_element_type=jnp.float32
        )
        rows = pl.ds(t * BM, BM)
        dh_scr[rows, :] = dh.astype(jnp.bfloat16)
        y1_scr[rows, :] = (
            (1.0 - ALP) * h_ref[rows, :] + (ALP * LAM) * ah + ALP * dh
        )

    @pl.when(t >= P)
    def _():
        i, c = _decode(t - P)
        y1c = y1_scr[pl.ds(c * CK, CK), :]
        gid = c * CK + jax.lax.broadcasted_iota(jnp.int32, (CK, 1), 0)
        y1m = jnp.where(gid >= i * BM, y1c, 0.0).astype(jnp.bfloat16)
        rows = pl.ds(i * BM, BM)
        acc2_scr[rows, :] += jnp.dot(
            a2_ref[...].astype(jnp.bfloat16), y1m, preferred_element_type=jnp.float32
        )

        @pl.when(c == NC - 1)
        def _():
            y2 = (
                (1.0 - ALP) * y1_scr[rows, :]
                + (ALP * LAM) * acc2_scr[rows, :]
                + ALP * dh_scr[rows, :].astype(jnp.float32)
            )
            z = jnp.maximum(y2, 0.0)
            out_ref[...] = (
                jnp.dot(z, w2_ref[...], preferred_element_type=jnp.float32)
                + b2_ref[...]
            )


def _a1_map(t):
    return (jnp.minimum(t, P - 1), 0)


def _a2_map(t):
    i, c = _decode(jnp.maximum(t - P, 0))
    return (i, c)


def _out_map(t):
    i, _ = _decode(jnp.maximum(t - P, 0))
    return (i, 0)


def kernel(x, sem_adj, norm_diag, W_bef, b_bef, W_aft, b_aft):
    h = pl.pallas_call(
        _h_kernel,
        out_shape=jax.ShapeDtypeStruct((N, HIDDEN_D), jnp.float32),
    )(x, W_bef, b_bef.reshape(1, HIDDEN_D))

    out = pl.pallas_call(
        _fused_kernel,
        grid=(P + SWEEP2_STEPS,),
        in_specs=[
            pl.BlockSpec((N, HIDDEN_D), lambda t: (0, 0)),
            pl.BlockSpec((BM, N), _a1_map),
            pl.BlockSpec((BM, N), _a1_map),
            pl.BlockSpec((BM, CK), _a2_map),
            pl.BlockSpec((HIDDEN_D, OUTPUT_D), lambda t: (0, 0)),
            pl.BlockSpec((1, OUTPUT_D), lambda t: (0, 0)),
        ],
        out_specs=pl.BlockSpec((BM, OUTPUT_D), _out_map),
        out_shape=jax.ShapeDtypeStruct((N, OUTPUT_D), jnp.float32),
        scratch_shapes=[
            pltpu.VMEM((N, HIDDEN_D), jnp.float32),
            pltpu.VMEM((N, HIDDEN_D), jnp.bfloat16),
            pltpu.VMEM((N, HIDDEN_D), jnp.float32),
        ],
        compiler_params=pltpu.CompilerParams(
            dimension_semantics=("arbitrary",),
        ),
    )(h, sem_adj, norm_diag, sem_adj, W_aft, b_aft.reshape(1, OUTPUT_D))

    return out


# speedup vs baseline: 1.2446x; 1.0961x over previous
import jax
import jax.numpy as jnp
from jax.experimental import pallas as pl
from jax.experimental.pallas import tpu as pltpu

N = 8192
INPUT_D = 256
HIDDEN_D = 128
OUTPUT_D = 64
ALP = 0.5
LAM = 1.0

BM = 256
P = N // BM
HP = P // 2
T = N // 2
SWEEP2_STEPS = 2 * HP + HP


def _decode(u):
    top = u < 2 * HP
    i = jnp.where(top, u // 2, HP + (u - 2 * HP))
    c = jnp.where(top, u % 2, 1)
    return i, c


def _h_kernel(x_ref, w_ref, b_ref, h_ref):
    h_ref[...] = (
        jnp.dot(x_ref[...], w_ref[...], preferred_element_type=jnp.float32)
        + b_ref[...]
    )


def _fused_kernel(
    h_ref, a1_ref, d_ref, a2_ref, w2_ref, b2_ref,
    out_ref, y1_scr, dh_scr, acc2_scr,
):
    t = pl.program_id(0)

    @pl.when(t == 0)
    def _():
        acc2_scr[...] = jnp.zeros_like(acc2_scr)

    @pl.when(t < P)
    def _():
        h = h_ref[...].astype(jnp.bfloat16)
        a = a1_ref[...].astype(jnp.bfloat16)
        d = d_ref[...].astype(jnp.bfloat16)
        ah = jnp.dot(a, h, preferred_element_type=jnp.float32)
        dh = jnp.dot(d, h, preferred_element_type=jnp.float32)
        rows = pl.ds(t * BM, BM)
        dh_scr[rows, :] = dh.astype(jnp.bfloat16)
        y1_scr[rows, :] = (
            (1.0 - ALP) * h_ref[rows, :] + (ALP * LAM) * ah + ALP * dh
        )

        @pl.when(t >= HP)
        def _():
            acc2_scr[rows, :] += jnp.dot(
                a[:, :T],
                y1_scr[pl.ds(0, T), :].astype(jnp.bfloat16),
                preferred_element_type=jnp.float32,
            )

    @pl.when(t >= P)
    def _():
        i, c = _decode(t - P)
        y1c = y1_scr[pl.ds(c * T, T), :].astype(jnp.bfloat16)
        rows = pl.ds(i * BM, BM)
        acc2_scr[rows, :] += jnp.dot(
            a2_ref[...].astype(jnp.bfloat16), y1c,
            preferred_element_type=jnp.float32,
        )

        @pl.when(c == 1)
        def _():
            y2 = (
                (1.0 - ALP) * y1_scr[rows, :]
                + (ALP * LAM) * acc2_scr[rows, :]
                + ALP * dh_scr[rows, :].astype(jnp.float32)
            )
            z = jnp.maximum(y2, 0.0)
            out_ref[...] = (
                jnp.dot(z, w2_ref[...], preferred_element_type=jnp.float32)
                + b2_ref[...]
            )


def _a1_map(t):
    return (jnp.minimum(t, P - 1), 0)


def _a2_map(t):
    i, c = _decode(jnp.maximum(t - P, 0))
    return (i, c)


def _out_map(t):
    i, _ = _decode(jnp.maximum(t - P, 0))
    return (i, 0)


def kernel(x, sem_adj, norm_diag, W_bef, b_bef, W_aft, b_aft):
    h = pl.pallas_call(
        _h_kernel,
        out_shape=jax.ShapeDtypeStruct((N, HIDDEN_D), jnp.float32),
    )(x, W_bef, b_bef.reshape(1, HIDDEN_D))

    out = pl.pallas_call(
        _fused_kernel,
        grid=(P + SWEEP2_STEPS,),
        in_specs=[
            pl.BlockSpec((N, HIDDEN_D), lambda t: (0, 0)),
            pl.BlockSpec((BM, N), _a1_map),
            pl.BlockSpec((BM, N), _a1_map),
            pl.BlockSpec((BM, T), _a2_map),
            pl.BlockSpec((HIDDEN_D, OUTPUT_D), lambda t: (0, 0)),
            pl.BlockSpec((1, OUTPUT_D), lambda t: (0, 0)),
        ],
        out_specs=pl.BlockSpec((BM, OUTPUT_D), _out_map),
        out_shape=jax.ShapeDtypeStruct((N, OUTPUT_D), jnp.float32),
        scratch_shapes=[
            pltpu.VMEM((N, HIDDEN_D), jnp.float32),
            pltpu.VMEM((N, HIDDEN_D), jnp.bfloat16),
            pltpu.VMEM((N, HIDDEN_D), jnp.float32),
        ],
        compiler_params=pltpu.CompilerParams(
            dimension_semantics=("arbitrary",),
        ),
    )(h, sem_adj, norm_diag, sem_adj, W_aft, b_aft.reshape(1, OUTPUT_D))

    return out


# speedup vs baseline: 1.3069x; 1.0500x over previous
import jax
import jax.numpy as jnp
from jax.experimental import pallas as pl
from jax.experimental.pallas import tpu as pltpu

N = 8192
INPUT_D = 256
HIDDEN_D = 128
OUTPUT_D = 64
ALP = 0.5
LAM = 1.0

BM = 256
P = N // BM
HP = P // 2
T = N // 2


def _h_kernel(x_ref, w_ref, b_ref, h_ref):
    h_ref[...] = (
        jnp.dot(x_ref[...], w_ref[...], preferred_element_type=jnp.float32)
        + b_ref[...]
    )


def _fused_kernel(
    h_ref, a1_ref, d_ref, a2_ref, w2_ref, b2_ref,
    out_ref, y1_scr, dh_scr, acc2_scr,
):
    t = pl.program_id(0)

    @pl.when(t == 0)
    def _():
        acc2_scr[...] = jnp.zeros_like(acc2_scr)

    @pl.when(t < P)
    def _():
        h = h_ref[...].astype(jnp.bfloat16)
        a = a1_ref[...].astype(jnp.bfloat16)
        d = d_ref[...].astype(jnp.bfloat16)
        ah = jnp.dot(a, h, preferred_element_type=jnp.float32)
        dh = jnp.dot(d, h, preferred_element_type=jnp.float32)
        rows = pl.ds(t * BM, BM)
        dh_scr[rows, :] = dh.astype(jnp.bfloat16)
        y1_scr[rows, :] = (
            (1.0 - ALP) * h_ref[rows, :] + (ALP * LAM) * ah + ALP * dh
        )

        @pl.when(t >= HP)
        def _():
            acc2_scr[rows, :] += jnp.dot(
                a[:, :T],
                y1_scr[pl.ds(0, T), :].astype(jnp.bfloat16),
                preferred_element_type=jnp.float32,
            )

    @pl.when(t >= P)
    def _():
        i = t - P
        rows = pl.ds(i * BM, BM)

        @pl.when(t < P + HP)
        def _():
            acc2_scr[rows, :] += jnp.dot(
                a1_ref[...].astype(jnp.bfloat16),
                y1_scr[...].astype(jnp.bfloat16),
                preferred_element_type=jnp.float32,
            )

        @pl.when(t >= P + HP)
        def _():
            acc2_scr[rows, :] += jnp.dot(
                a2_ref[...].astype(jnp.bfloat16),
                y1_scr[pl.ds(T, T), :].astype(jnp.bfloat16),
                preferred_element_type=jnp.float32,
            )

        y2 = (
            (1.0 - ALP) * y1_scr[rows, :]
            + (ALP * LAM) * acc2_scr[rows, :]
            + ALP * dh_scr[rows, :].astype(jnp.float32)
        )
        z = jnp.maximum(y2, 0.0)
        out_ref[...] = (
            jnp.dot(z, w2_ref[...], preferred_element_type=jnp.float32)
            + b2_ref[...]
        )


def _a1_map(t):
    return (jnp.where(t < P, t, jnp.where(t < P + HP, t - P, HP - 1)), 0)


def _a2_map(t):
    return (jnp.maximum(t - P, HP), 1)


def _out_map(t):
    return (jnp.maximum(t - P, 0), 0)


def kernel(x, sem_adj, norm_diag, W_bef, b_bef, W_aft, b_aft):
    h = pl.pallas_call(
        _h_kernel,
        out_shape=jax.ShapeDtypeStruct((N, HIDDEN_D), jnp.float32),
    )(x, W_bef, b_bef.reshape(1, HIDDEN_D))

    out = pl.pallas_call(
        _fused_kernel,
        grid=(2 * P,),
        in_specs=[
            pl.BlockSpec((N, HIDDEN_D), lambda t: (0, 0)),
            pl.BlockSpec((BM, N), _a1_map),
            pl.BlockSpec((BM, N), lambda t: (jnp.minimum(t, P - 1), 0)),
            pl.BlockSpec((BM, T), _a2_map),
            pl.BlockSpec((HIDDEN_D, OUTPUT_D), lambda t: (0, 0)),
            pl.BlockSpec((1, OUTPUT_D), lambda t: (0, 0)),
        ],
        out_specs=pl.BlockSpec((BM, OUTPUT_D), _out_map),
        out_shape=jax.ShapeDtypeStruct((N, OUTPUT_D), jnp.float32),
        scratch_shapes=[
            pltpu.VMEM((N, HIDDEN_D), jnp.float32),
            pltpu.VMEM((N, HIDDEN_D), jnp.bfloat16),
            pltpu.VMEM((N, HIDDEN_D), jnp.float32),
        ],
        compiler_params=pltpu.CompilerParams(
            dimension_semantics=("arbitrary",),
        ),
    )(h, sem_adj, norm_diag, sem_adj, W_aft, b_aft.reshape(1, OUTPUT_D))

    return out


# speedup vs baseline: 1.3122x; 1.0041x over previous
import jax
import jax.numpy as jnp
from jax.experimental import pallas as pl
from jax.experimental.pallas import tpu as pltpu

N = 8192
INPUT_D = 256
HIDDEN_D = 128
OUTPUT_D = 64
ALP = 0.5
LAM = 1.0

BM = 256
P = N // BM
HP = P // 2
T = N // 2


def _h_kernel(x_ref, w_ref, b_ref, h_ref):
    h_ref[...] = (
        jnp.dot(x_ref[...], w_ref[...], preferred_element_type=jnp.float32)
        + b_ref[...]
    ).astype(jnp.bfloat16)


def _fused_kernel(
    h_ref, a1_ref, d_ref, a2_ref, w2_ref, b2_ref,
    out_ref, y1_scr, dh_scr, acc2_scr,
):
    t = pl.program_id(0)

    @pl.when(t == 0)
    def _():
        acc2_scr[...] = jnp.zeros_like(acc2_scr)

    @pl.when(t < P)
    def _():
        h = h_ref[...]
        a = a1_ref[...].astype(jnp.bfloat16)
        d = d_ref[...].astype(jnp.bfloat16)
        ah = jnp.dot(a, h, preferred_element_type=jnp.float32)
        dh = jnp.dot(d, h, preferred_element_type=jnp.float32)
        rows = pl.ds(t * BM, BM)
        dh_scr[rows, :] = dh.astype(jnp.bfloat16)
        y1_scr[rows, :] = (
            (1.0 - ALP) * h_ref[rows, :].astype(jnp.float32)
            + (ALP * LAM) * ah + ALP * dh
        ).astype(jnp.bfloat16)

        @pl.when(t >= HP)
        def _():
            acc2_scr[rows, :] += jnp.dot(
                a[:, :T], y1_scr[pl.ds(0, T), :],
                preferred_element_type=jnp.float32,
            )

    @pl.when(t >= P)
    def _():
        i = t - P
        rows = pl.ds(i * BM, BM)

        @pl.when(t < P + HP)
        def _():
            acc2_scr[rows, :] += jnp.dot(
                a1_ref[...].astype(jnp.bfloat16), y1_scr[...],
                preferred_element_type=jnp.float32,
            )

        @pl.when(t >= P + HP)
        def _():
            acc2_scr[rows, :] += jnp.dot(
                a2_ref[...].astype(jnp.bfloat16), y1_scr[pl.ds(T, T), :],
                preferred_element_type=jnp.float32,
            )

        y2 = (
            (1.0 - ALP) * y1_scr[rows, :].astype(jnp.float32)
            + (ALP * LAM) * acc2_scr[rows, :]
            + ALP * dh_scr[rows, :].astype(jnp.float32)
        )
        z = jnp.maximum(y2, 0.0)
        out_ref[...] = (
            jnp.dot(z, w2_ref[...], preferred_element_type=jnp.float32)
            + b2_ref[...]
        )


def _a1_map(t):
    return (jnp.where(t < P, t, jnp.where(t < P + HP, t - P, HP - 1)), 0)


def _a2_map(t):
    return (jnp.maximum(t - P, HP), 1)


def _out_map(t):
    return (jnp.maximum(t - P, 0), 0)


def kernel(x, sem_adj, norm_diag, W_bef, b_bef, W_aft, b_aft):
    h = pl.pallas_call(
        _h_kernel,
        out_shape=jax.ShapeDtypeStruct((N, HIDDEN_D), jnp.bfloat16),
    )(x, W_bef, b_bef.reshape(1, HIDDEN_D))

    out = pl.pallas_call(
        _fused_kernel,
        grid=(2 * P,),
        in_specs=[
            pl.BlockSpec((N, HIDDEN_D), lambda t: (0, 0)),
            pl.BlockSpec((BM, N), _a1_map),
            pl.BlockSpec((BM, N), lambda t: (jnp.minimum(t, P - 1), 0)),
            pl.BlockSpec((BM, T), _a2_map),
            pl.BlockSpec((HIDDEN_D, OUTPUT_D), lambda t: (0, 0)),
            pl.BlockSpec((1, OUTPUT_D), lambda t: (0, 0)),
        ],
        out_specs=pl.BlockSpec((BM, OUTPUT_D), _out_map),
        out_shape=jax.ShapeDtypeStruct((N, OUTPUT_D), jnp.float32),
        scratch_shapes=[
            pltpu.VMEM((N, HIDDEN_D), jnp.bfloat16),
            pltpu.VMEM((N, HIDDEN_D), jnp.bfloat16),
            pltpu.VMEM((N, HIDDEN_D), jnp.float32),
        ],
        compiler_params=pltpu.CompilerParams(
            dimension_semantics=("arbitrary",),
        ),
    )(h, sem_adj, norm_diag, sem_adj, W_aft, b_aft.reshape(1, OUTPUT_D))

    return out


# speedup vs baseline: 1.3227x; 1.0080x over previous
import jax
import jax.numpy as jnp
from jax.experimental import pallas as pl
from jax.experimental.pallas import tpu as pltpu

N = 8192
INPUT_D = 256
HIDDEN_D = 128
OUTPUT_D = 64
ALP = 0.5
LAM = 1.0

BM = 256
P = N // BM
Q = N // 4
BPB = P // 4


def _h_kernel(x_ref, w_ref, b_ref, h_ref):
    h_ref[...] = (
        jnp.dot(x_ref[...], w_ref[...], preferred_element_type=jnp.float32)
        + b_ref[...]
    ).astype(jnp.bfloat16)


def _row_mask(y1c, col0, i):
    gid = col0 + jax.lax.broadcasted_iota(jnp.int32, (y1c.shape[0], 1), 0)
    return jnp.where(gid >= i * BM, y1c, jnp.bfloat16(0.0))


def _fused_kernel(
    h_ref, a1_ref, d_ref, aq_ref, ah_ref, w2_ref, b2_ref,
    out_ref, y1_scr, dh_scr, acc2_scr,
):
    t = pl.program_id(0)

    @pl.when(t == 0)
    def _():
        y1_scr[...] = jnp.zeros_like(y1_scr)

    @pl.when(t < P)
    def _():
        h = h_ref[...]
        a = a1_ref[...].astype(jnp.bfloat16)
        d = d_ref[...].astype(jnp.bfloat16)
        ah = jnp.dot(a, h, preferred_element_type=jnp.float32)
        dh = jnp.dot(d, h, preferred_element_type=jnp.float32)
        acc2_scr[pl.ds(t * BM, BM), :] = jnp.dot(
            a, y1_scr[...], preferred_element_type=jnp.float32
        ).astype(jnp.bfloat16)
        rows = pl.ds(t * BM, BM)
        dh_scr[rows, :] = dh.astype(jnp.bfloat16)
        y1_scr[rows, :] = (
            (1.0 - ALP) * h_ref[rows, :].astype(jnp.float32)
            + (ALP * LAM) * ah + ALP * dh
        ).astype(jnp.bfloat16)

    @pl.when(t >= P)
    def _():
        i = t - P
        b = i // BPB
        rows = pl.ds(i * BM, BM)
        acc = acc2_scr[rows, :].astype(jnp.float32)

        @pl.when(b == 0)
        def _():
            p0 = jnp.dot(
                a1_ref[:, :Q].astype(jnp.bfloat16),
                _row_mask(y1_scr[pl.ds(0, Q), :], 0, i),
                preferred_element_type=jnp.float32,
            )
            p1 = jnp.dot(
                a1_ref[:, Q:].astype(jnp.bfloat16),
                y1_scr[pl.ds(Q, 3 * Q), :],
                preferred_element_type=jnp.float32,
            )
            acc2_scr[rows, :] = (acc + p0 + p1).astype(jnp.bfloat16)

        @pl.when(b == 1)
        def _():
            p0 = jnp.dot(
                aq_ref[...].astype(jnp.bfloat16),
                _row_mask(y1_scr[pl.ds(Q, Q), :], Q, i),
                preferred_element_type=jnp.float32,
            )
            p1 = jnp.dot(
                ah_ref[...].astype(jnp.bfloat16),
                y1_scr[pl.ds(2 * Q, 2 * Q), :],
                preferred_element_type=jnp.float32,
            )
            acc2_scr[rows, :] = (acc + p0 + p1).astype(jnp.bfloat16)

        @pl.when(b == 2)
        def _():
            p0 = jnp.dot(
                ah_ref[...].astype(jnp.bfloat16),
                _row_mask(y1_scr[pl.ds(2 * Q, 2 * Q), :], 2 * Q, i),
                preferred_element_type=jnp.float32,
            )
            acc2_scr[rows, :] = (acc + p0).astype(jnp.bfloat16)

        @pl.when(b == 3)
        def _():
            p0 = jnp.dot(
                aq_ref[...].astype(jnp.bfloat16),
                _row_mask(y1_scr[pl.ds(3 * Q, Q), :], 3 * Q, i),
                preferred_element_type=jnp.float32,
            )
            acc2_scr[rows, :] = (acc + p0).astype(jnp.bfloat16)

        y2 = (
            (1.0 - ALP) * y1_scr[rows, :].astype(jnp.float32)
            + (ALP * LAM) * acc2_scr[rows, :].astype(jnp.float32)
            + ALP * dh_scr[rows, :].astype(jnp.float32)
        )
        z = jnp.maximum(y2, 0.0)
        out_ref[...] = (
            jnp.dot(z, w2_ref[...], preferred_element_type=jnp.float32)
            + b2_ref[...]
        )


def _a1_map(t):
    return (jnp.where(t < P, t, jnp.minimum(t - P, BPB - 1)), 0)


def _aq_map(t):
    i = t - P
    useful = jnp.logical_or(
        jnp.logical_and(i >= BPB, i < 2 * BPB), i >= 3 * BPB
    )
    row = jnp.where(useful, i, BPB)
    col = jnp.where(i >= 3 * BPB, 3, 1)
    return (jnp.where(t < P, BPB, row), jnp.where(t < P, 1, col))


def _ah_map(t):
    i = t - P
    useful = jnp.logical_and(i >= BPB, i < 3 * BPB)
    return (jnp.where(useful, i, BPB), 1)


def _out_map(t):
    return (jnp.maximum(t - P, 0), 0)


def kernel(x, sem_adj, norm_diag, W_bef, b_bef, W_aft, b_aft):
    h = pl.pallas_call(
        _h_kernel,
        out_shape=jax.ShapeDtypeStruct((N, HIDDEN_D), jnp.bfloat16),
    )(x, W_bef, b_bef.reshape(1, HIDDEN_D))

    out = pl.pallas_call(
        _fused_kernel,
        grid=(2 * P,),
        in_specs=[
            pl.BlockSpec((N, HIDDEN_D), lambda t: (0, 0)),
            pl.BlockSpec((BM, N), _a1_map),
            pl.BlockSpec((BM, N), lambda t: (jnp.minimum(t, P - 1), 0)),
            pl.BlockSpec((BM, Q), _aq_map),
            pl.BlockSpec((BM, 2 * Q), _ah_map),
            pl.BlockSpec((HIDDEN_D, OUTPUT_D), lambda t: (0, 0)),
            pl.BlockSpec((1, OUTPUT_D), lambda t: (0, 0)),
        ],
        out_specs=pl.BlockSpec((BM, OUTPUT_D), _out_map),
        out_shape=jax.ShapeDtypeStruct((N, OUTPUT_D), jnp.float32),
        scratch_shapes=[
            pltpu.VMEM((N, HIDDEN_D), jnp.bfloat16),
            pltpu.VMEM((N, HIDDEN_D), jnp.bfloat16),
            pltpu.VMEM((N, HIDDEN_D), jnp.bfloat16),
        ],
        compiler_params=pltpu.CompilerParams(
            dimension_semantics=("arbitrary",),
        ),
    )(h, sem_adj, norm_diag, sem_adj, sem_adj, W_aft, b_aft.reshape(1, OUTPUT_D))

    return out
